# baseline (speedup 1.0000x reference)
import functools

import numpy as np

import jax
import jax.numpy as jnp
from jax import lax
from jax.experimental import pallas as pl
from jax.experimental.pallas import tpu as pltpu

_Q = 0.4
_ALPHA = 0.0
_LANES = 128


def _pool_head_kernel(lc_ref, lp_ref, x_ref, m_ref, w_ref, aux_ref, out_ref,
                      *, num_cat, num_pol):
    i = pl.program_id(0)
    x = x_ref[...]
    bb, S, H = x.shape
    m = m_ref[...].astype(jnp.float32)

    rowi = lax.broadcasted_iota(jnp.int32, (bb, S), 0)
    sel = jnp.concatenate(
        [jnp.where(rowi == r, m, 0.0) for r in range(bb)], axis=1)
    pool = jnp.dot(sel.astype(jnp.bfloat16),
                   x.reshape(bb * S, H).astype(jnp.bfloat16),
                   preferred_element_type=jnp.float32)

    den = jnp.maximum(jnp.sum(m, axis=1, keepdims=True), 1.0)
    se = pool / den

    bias = aux_ref[0:1, :]
    lw = aux_ref[1:2, :]
    logits = jnp.dot(se, w_ref[...],
                     preferred_element_type=jnp.float32) + bias

    col = lax.broadcasted_iota(jnp.int32, logits.shape, 1)
    rowi2 = lax.broadcasted_iota(jnp.int32, logits.shape, 0)
    onehot = jnp.zeros(logits.shape, jnp.float32)
    for r in range(bb):
        lc = lc_ref[i * bb + r]
        lp = lp_ref[i * bb + r] + num_cat
        hit = (jnp.logical_or(col == lc, col == lp)).astype(jnp.float32)
        onehot = jnp.where(rowi2 == r, hit, onehot)
    head_cat = (col < num_cat).astype(jnp.float32)
    head_pol = jnp.logical_and(col >= num_cat,
                               col < num_cat + num_pol).astype(jnp.float32)

    def lq_term(head):
        z = jnp.where(head > 0.0, logits, -1e30)
        e = jnp.exp(z - jnp.max(z, axis=-1, keepdims=True)) * head
        p = e / jnp.sum(e, axis=-1, keepdims=True)
        yq = jnp.maximum(jnp.sum(p * onehot, axis=-1, keepdims=True), 1e-12)
        lq = (1.0 - jnp.exp(_Q * jnp.log(yq))) / _Q
        wg = jnp.sum(lw * onehot * head, axis=-1, keepdims=True)
        return _ALPHA * lq + (1.0 - _ALPHA) * lq * wg

    t_cat = lq_term(head_cat)
    t_pol = lq_term(head_pol)

    out_ref[...] = jnp.where(col == _LANES - 2, t_cat,
                             jnp.where(col == _LANES - 1, t_pol, logits))


def _round_up(n, m):
    return -(-n // m) * m


def kernel(x, attention_mask, w_cat, b_cat, w_pol, b_pol,
           aspect_weights, sentiment_weights, labels_cat, labels_pol):
    B, S, H = x.shape
    num_cat = w_cat.shape[1]
    num_pol = w_pol.shape[1]
    C = num_cat + num_pol
    assert C + 2 <= _LANES
    f32 = jnp.float32

    bb = 64
    B_pad = _round_up(B, bb)
    nb = B_pad // bb

    x_p = x
    mask = attention_mask.astype(jnp.int32)
    if B_pad != B:
        x_p = jnp.concatenate(
            [x_p, jnp.zeros((B_pad - B, S, H), x.dtype)], axis=0)
        mask = jnp.concatenate(
            [mask, jnp.zeros((B_pad - B, S), jnp.int32)], axis=0)

    w_all = jnp.concatenate(
        [w_cat.astype(f32), w_pol.astype(f32),
         jnp.zeros((H, _LANES - C), f32)], axis=1)

    lw_cat = jax.nn.softmax(-jnp.log(aspect_weights.astype(f32)))
    lw_pol = jax.nn.softmax(-jnp.log(sentiment_weights.astype(f32)))
    zpad = jnp.zeros((_LANES - C,), f32)
    aux = jnp.concatenate([
        jnp.concatenate([b_cat.astype(f32), b_pol.astype(f32), zpad])[None],
        jnp.concatenate([lw_cat, lw_pol, zpad])[None],
        jnp.zeros((6, _LANES), f32)], axis=0)

    lc = labels_cat.astype(jnp.int32)
    lp = labels_pol.astype(jnp.int32)
    if B_pad != B:
        zpad_i = jnp.zeros((B_pad - B,), jnp.int32)
        lc = jnp.concatenate([lc, zpad_i])
        lp = jnp.concatenate([lp, zpad_i])

    kernel_fn = functools.partial(_pool_head_kernel,
                                  num_cat=num_cat, num_pol=num_pol)

    tile_bytes = bb * S * H * 4
    vmem_limit = int(min(2 * tile_bytes + (16 << 20), 64 << 20))

    out = pl.pallas_call(
        kernel_fn,
        out_shape=jax.ShapeDtypeStruct((B_pad, _LANES), f32),
        grid_spec=pltpu.PrefetchScalarGridSpec(
            num_scalar_prefetch=2,
            grid=(nb,),
            in_specs=[
                pl.BlockSpec((bb, S, H), lambda i, lc_r, lp_r: (i, 0, 0)),
                pl.BlockSpec((bb, S), lambda i, lc_r, lp_r: (i, 0)),
                pl.BlockSpec((H, _LANES), lambda i, lc_r, lp_r: (0, 0)),
                pl.BlockSpec((8, _LANES), lambda i, lc_r, lp_r: (0, 0)),
            ],
            out_specs=pl.BlockSpec((bb, _LANES), lambda i, lc_r, lp_r: (i, 0)),
        ),
        compiler_params=pltpu.CompilerParams(
            dimension_semantics=("parallel",),
            vmem_limit_bytes=vmem_limit),
    )(lc, lp, x_p, mask, w_all, aux)

    loss = (jnp.square(jnp.sum(out[:B, _LANES - 2]) / B) +
            jnp.square(jnp.sum(out[:B, _LANES - 1]) / B))
    return (loss, out[:B, :num_cat], out[:B, num_cat:C])


# speedup vs baseline: 1.0944x; 1.0944x over previous
import functools

import numpy as np

import jax
import jax.numpy as jnp
from jax import lax
from jax.experimental import pallas as pl
from jax.experimental.pallas import tpu as pltpu

_Q = 0.4
_ALPHA = 0.0
_LANES = 128


def _pool_head_kernel(lc_ref, lp_ref, xa_ref, xb_ref, m_ref, w_ref, aux_ref,
                      out_ref, *, num_cat, num_pol):
    i = pl.program_id(0)
    m = m_ref[...].astype(jnp.float32)
    bb, S = m.shape
    S2 = S // 2

    rowi = lax.broadcasted_iota(jnp.int32, (bb, S2), 0)

    def half_pool(x_half, m_half):
        sel = jnp.concatenate(
            [jnp.where(rowi == r, m_half, 0.0) for r in range(bb)], axis=1)
        _, _, H = x_half.shape
        return jnp.dot(sel.astype(jnp.bfloat16),
                       x_half.reshape(bb * S2, H).astype(jnp.bfloat16),
                       preferred_element_type=jnp.float32)

    pool = (half_pool(xa_ref[...], m[:, :S2]) +
            half_pool(xb_ref[...], m[:, S2:]))

    den = jnp.maximum(jnp.sum(m, axis=1, keepdims=True), 1.0)
    se = pool / den

    bias = aux_ref[0:1, :]
    lw = aux_ref[1:2, :]
    logits = jnp.dot(se, w_ref[...],
                     preferred_element_type=jnp.float32) + bias

    col = lax.broadcasted_iota(jnp.int32, logits.shape, 1)
    rowi2 = lax.broadcasted_iota(jnp.int32, logits.shape, 0)
    onehot = jnp.zeros(logits.shape, jnp.float32)
    for r in range(bb):
        lc = lc_ref[i * bb + r]
        lp = lp_ref[i * bb + r] + num_cat
        hit = (jnp.logical_or(col == lc, col == lp)).astype(jnp.float32)
        onehot = jnp.where(rowi2 == r, hit, onehot)
    head_cat = (col < num_cat).astype(jnp.float32)
    head_pol = jnp.logical_and(col >= num_cat,
                               col < num_cat + num_pol).astype(jnp.float32)

    def lq_term(head):
        z = jnp.where(head > 0.0, logits, -1e30)
        e = jnp.exp(z - jnp.max(z, axis=-1, keepdims=True)) * head
        p = e / jnp.sum(e, axis=-1, keepdims=True)
        yq = jnp.maximum(jnp.sum(p * onehot, axis=-1, keepdims=True), 1e-12)
        lq = (1.0 - jnp.exp(_Q * jnp.log(yq))) / _Q
        wg = jnp.sum(lw * onehot * head, axis=-1, keepdims=True)
        return _ALPHA * lq + (1.0 - _ALPHA) * lq * wg

    t_cat = lq_term(head_cat)
    t_pol = lq_term(head_pol)

    out_ref[...] = jnp.where(col == _LANES - 2, t_cat,
                             jnp.where(col == _LANES - 1, t_pol, logits))


def _round_up(n, m):
    return -(-n // m) * m


def kernel(x, attention_mask, w_cat, b_cat, w_pol, b_pol,
           aspect_weights, sentiment_weights, labels_cat, labels_pol):
    B, S, H = x.shape
    num_cat = w_cat.shape[1]
    num_pol = w_pol.shape[1]
    C = num_cat + num_pol
    assert C + 2 <= _LANES
    f32 = jnp.float32

    bb = 32
    B_pad = _round_up(B, bb)
    nb = B_pad // bb

    x_p = x
    mask = attention_mask.astype(jnp.int32)
    if B_pad != B:
        x_p = jnp.concatenate(
            [x_p, jnp.zeros((B_pad - B, S, H), x.dtype)], axis=0)
        mask = jnp.concatenate(
            [mask, jnp.zeros((B_pad - B, S), jnp.int32)], axis=0)

    w_all = jnp.concatenate(
        [w_cat.astype(f32), w_pol.astype(f32),
         jnp.zeros((H, _LANES - C), f32)], axis=1)

    lw_cat = jax.nn.softmax(-jnp.log(aspect_weights.astype(f32)))
    lw_pol = jax.nn.softmax(-jnp.log(sentiment_weights.astype(f32)))
    zpad = jnp.zeros((_LANES - C,), f32)
    aux = jnp.concatenate([
        jnp.concatenate([b_cat.astype(f32), b_pol.astype(f32), zpad])[None],
        jnp.concatenate([lw_cat, lw_pol, zpad])[None],
        jnp.zeros((6, _LANES), f32)], axis=0)

    lc = labels_cat.astype(jnp.int32)
    lp = labels_pol.astype(jnp.int32)
    if B_pad != B:
        zpad_i = jnp.zeros((B_pad - B,), jnp.int32)
        lc = jnp.concatenate([lc, zpad_i])
        lp = jnp.concatenate([lp, zpad_i])

    kernel_fn = functools.partial(_pool_head_kernel,
                                  num_cat=num_cat, num_pol=num_pol)

    tile_bytes = bb * S * H * 4
    vmem_limit = int(min(2 * tile_bytes + (16 << 20), 64 << 20))

    out = pl.pallas_call(
        kernel_fn,
        out_shape=jax.ShapeDtypeStruct((B_pad, _LANES), f32),
        grid_spec=pltpu.PrefetchScalarGridSpec(
            num_scalar_prefetch=2,
            grid=(nb,),
            in_specs=[
                pl.BlockSpec((bb, S // 2, H), lambda i, lc_r, lp_r: (i, 0, 0)),
                pl.BlockSpec((bb, S // 2, H), lambda i, lc_r, lp_r: (i, 1, 0)),
                pl.BlockSpec((bb, S), lambda i, lc_r, lp_r: (i, 0)),
                pl.BlockSpec((H, _LANES), lambda i, lc_r, lp_r: (0, 0)),
                pl.BlockSpec((8, _LANES), lambda i, lc_r, lp_r: (0, 0)),
            ],
            out_specs=pl.BlockSpec((bb, _LANES), lambda i, lc_r, lp_r: (i, 0)),
        ),
        compiler_params=pltpu.CompilerParams(
            dimension_semantics=("parallel",),
            vmem_limit_bytes=vmem_limit),
    )(lc, lp, x_p, x_p, mask, w_all, aux)

    loss = (jnp.square(jnp.sum(out[:B, _LANES - 2]) / B) +
            jnp.square(jnp.sum(out[:B, _LANES - 1]) / B))
    return (loss, out[:B, :num_cat], out[:B, num_cat:C])


# speedup vs baseline: 1.2156x; 1.1107x over previous
import functools

import jax
import jax.numpy as jnp
from jax import lax
from jax.experimental import pallas as pl
from jax.experimental.pallas import tpu as pltpu

_Q = 0.4
_ALPHA = 0.0


def _scalars_to_row(ref, n):
    col = lax.broadcasted_iota(jnp.int32, (1, n), 1)
    row = jnp.zeros((1, n), jnp.float32)
    for k in range(n):
        row = jnp.where(col == k, ref[k], row)
    return row


def _softmax_neg_log(w_row):
    z = -jnp.log(w_row)
    e = jnp.exp(z - jnp.max(z, axis=-1, keepdims=True))
    return e / jnp.sum(e, axis=-1, keepdims=True)


def _lq_terms(logits, onehot, lw):
    e = jnp.exp(logits - jnp.max(logits, axis=-1, keepdims=True))
    p = e / jnp.sum(e, axis=-1, keepdims=True)
    yq = jnp.maximum(jnp.sum(p * onehot, axis=-1, keepdims=True), 1e-12)
    lq = (1.0 - jnp.exp(_Q * jnp.log(yq))) / _Q
    wg = jnp.sum(lw * onehot, axis=-1, keepdims=True)
    return _ALPHA * lq + (1.0 - _ALPHA) * lq * wg


def _fused_kernel(bc_ref, bp_ref, aw_ref, sw_ref, lc_ref, lp_ref,
                  xa_ref, xb_ref, m_ref, wc_ref, wp_ref,
                  oc_ref, op_ref, t_ref, *, num_cat, num_pol):
    i = pl.program_id(0)
    m = m_ref[...].astype(jnp.float32)
    bb, S = m.shape
    S2 = S // 2

    rowi = lax.broadcasted_iota(jnp.int32, (bb, S2), 0)

    def half_pool(x_half, m_half):
        sel = jnp.concatenate(
            [jnp.where(rowi == r, m_half, 0.0) for r in range(bb)], axis=1)
        _, _, H = x_half.shape
        return jnp.dot(sel.astype(jnp.bfloat16),
                       x_half.reshape(bb * S2, H).astype(jnp.bfloat16),
                       preferred_element_type=jnp.float32)

    pool = (half_pool(xa_ref[...], m[:, :S2]) +
            half_pool(xb_ref[...], m[:, S2:]))

    den = jnp.maximum(jnp.sum(m, axis=1, keepdims=True), 1.0)
    se = pool / den

    logits_c = jnp.dot(se, wc_ref[...],
                       preferred_element_type=jnp.float32)
    logits_p = jnp.dot(se, wp_ref[...],
                       preferred_element_type=jnp.float32)
    logits_c = logits_c + _scalars_to_row(bc_ref, num_cat)
    logits_p = logits_p + _scalars_to_row(bp_ref, num_pol)
    oc_ref[...] = logits_c
    op_ref[...] = logits_p

    lw_c = _softmax_neg_log(_scalars_to_row(aw_ref, num_cat))
    lw_p = _softmax_neg_log(_scalars_to_row(sw_ref, num_pol))

    col_c = lax.broadcasted_iota(jnp.int32, (bb, num_cat), 1)
    row_c = lax.broadcasted_iota(jnp.int32, (bb, num_cat), 0)
    col_p = lax.broadcasted_iota(jnp.int32, (bb, num_pol), 1)
    row_p = lax.broadcasted_iota(jnp.int32, (bb, num_pol), 0)
    oh_c = jnp.zeros((bb, num_cat), jnp.float32)
    oh_p = jnp.zeros((bb, num_pol), jnp.float32)
    for r in range(bb):
        oh_c = jnp.where(jnp.logical_and(row_c == r, col_c == lc_ref[i * bb + r]),
                         1.0, oh_c)
        oh_p = jnp.where(jnp.logical_and(row_p == r, col_p == lp_ref[i * bb + r]),
                         1.0, oh_p)

    t_cat = _lq_terms(logits_c, oh_c, lw_c)
    t_pol = jnp.broadcast_to(_lq_terms(logits_p, oh_p, lw_p), (bb, 1))
    col2 = lax.broadcasted_iota(jnp.int32, (bb, 2), 1)
    t_ref[...] = jnp.where(col2 == 0, t_cat, t_pol)


def _round_up(n, m):
    return -(-n // m) * m


def kernel(x, attention_mask, w_cat, b_cat, w_pol, b_pol,
           aspect_weights, sentiment_weights, labels_cat, labels_pol):
    B, S, H = x.shape
    num_cat = w_cat.shape[1]
    num_pol = w_pol.shape[1]
    f32 = jnp.float32

    bb = 32
    B_pad = _round_up(B, bb)
    nb = B_pad // bb

    x_p = x
    mask = attention_mask.astype(jnp.int32)
    lc = labels_cat.astype(jnp.int32)
    lp = labels_pol.astype(jnp.int32)
    if B_pad != B:
        x_p = jnp.concatenate(
            [x_p, jnp.zeros((B_pad - B, S, H), x.dtype)], axis=0)
        mask = jnp.concatenate(
            [mask, jnp.zeros((B_pad - B, S), jnp.int32)], axis=0)
        zpad_i = jnp.zeros((B_pad - B,), jnp.int32)
        lc = jnp.concatenate([lc, zpad_i])
        lp = jnp.concatenate([lp, zpad_i])

    kernel_fn = functools.partial(_fused_kernel,
                                  num_cat=num_cat, num_pol=num_pol)

    tile_bytes = bb * S * H * 4
    vmem_limit = int(min(2 * tile_bytes + (16 << 20), 64 << 20))

    logits_c, logits_p, terms = pl.pallas_call(
        kernel_fn,
        out_shape=(jax.ShapeDtypeStruct((B_pad, num_cat), f32),
                   jax.ShapeDtypeStruct((B_pad, num_pol), f32),
                   jax.ShapeDtypeStruct((B_pad, 2), f32)),
        grid_spec=pltpu.PrefetchScalarGridSpec(
            num_scalar_prefetch=6,
            grid=(nb,),
            in_specs=[
                pl.BlockSpec((bb, S // 2, H), lambda i, *_: (i, 0, 0)),
                pl.BlockSpec((bb, S // 2, H), lambda i, *_: (i, 1, 0)),
                pl.BlockSpec((bb, S), lambda i, *_: (i, 0)),
                pl.BlockSpec((H, num_cat), lambda i, *_: (0, 0)),
                pl.BlockSpec((H, num_pol), lambda i, *_: (0, 0)),
            ],
            out_specs=(
                pl.BlockSpec((bb, num_cat), lambda i, *_: (i, 0)),
                pl.BlockSpec((bb, num_pol), lambda i, *_: (i, 0)),
                pl.BlockSpec((bb, 2), lambda i, *_: (i, 0)),
            ),
        ),
        compiler_params=pltpu.CompilerParams(
            dimension_semantics=("arbitrary",),
            vmem_limit_bytes=vmem_limit),
    )(b_cat.astype(f32), b_pol.astype(f32),
      aspect_weights.astype(f32), sentiment_weights.astype(f32),
      lc, lp, x_p, x_p, mask, w_cat.astype(f32), w_pol.astype(f32))

    loss = (jnp.square(jnp.sum(terms[:B, 0]) / B) +
            jnp.square(jnp.sum(terms[:B, 1]) / B))
    return (loss, logits_c[:B], logits_p[:B])
